# baseline (speedup 1.0000x reference)
import functools
import itertools

import numpy as np
import jax
from jax import lax
import jax.numpy as jnp
from jax.experimental import pallas as pl
from jax.experimental.pallas import tpu as pltpu
from jax.experimental.pallas import tpu_sc as plsc

_DIM_X, _DIM_Y = 3, 3
_MAX_DIGIT = _DIM_X * _DIM_Y
_N = _MAX_DIGIT * _MAX_DIGIT
_NP = 96
_E = 16
_H = 96
_B = 64
_DEG = 20
_NE = _N * _DEG
_NEP = 1664
_ITERS = 2
_OUT = _MAX_DIGIT
_BBLK = 16
_NCHUNK = _B // _BBLK
_M = _BBLK * _NP


def _edge_table():
    rows = []
    for row in range(_MAX_DIGIT):
        for col in range(_MAX_DIGIT):
            s = {(row, i) for i in range(_MAX_DIGIT)}
            s |= {(i, col) for i in range(_MAX_DIGIT)}
            x_min = row // _DIM_X * _DIM_X
            y_min = col // _DIM_Y * _DIM_Y
            s |= set(itertools.product(range(x_min, x_min + _DIM_X),
                                       range(y_min, y_min + _DIM_Y)))
            s -= {(row, col)}
            rows.append(sorted(r * _MAX_DIGIT + c for r, c in s))
    return np.array(rows, dtype=np.int32)


def _gather_mats():
    edges = _edge_table()
    src = edges.reshape(-1)
    dst = np.repeat(np.arange(_N), _DEG)
    e = np.arange(_NE)
    g = np.zeros((_NEP, 2 * _NP + 8), np.float32)
    g[e, src] = 1.0
    g[e, _NP + dst] = 1.0
    g[e, 2 * _NP] = 1.0
    s = np.zeros((_NP, _NEP), np.float32)
    s[dst, e] = 1.0
    return g, s


_GCOMB, _SSUM = _gather_mats()


def _sc_embed_gather(table, idx, rows):
    info = plsc.get_sparse_core_info()
    nw = info.num_cores * info.num_subcores
    per = rows // nw
    mesh = plsc.VectorSubcoreMesh(core_axis_name="c", subcore_axis_name="s")

    @functools.partial(
        pl.kernel, mesh=mesh,
        out_type=jax.ShapeDtypeStruct((rows, 128), jnp.float32),
        scratch_types=[pltpu.VMEM((per,), jnp.int32),
                       pltpu.VMEM((per, 128), jnp.float32),
                       pltpu.SemaphoreType.DMA])
    def k(tab_hbm, idx_hbm, out_hbm, idx_v, rows_v, sem):
        wid = lax.axis_index("s") * info.num_cores + lax.axis_index("c")
        base = wid * per
        pltpu.sync_copy(idx_hbm.at[pl.ds(base, per)], idx_v)
        pltpu.async_copy(tab_hbm.at[idx_v], rows_v, sem).wait()
        pltpu.sync_copy(rows_v, out_hbm.at[pl.ds(base, per)])

    return k(table, idx)


def _body(emb_ref, c0_ref, w1e_ref, inb1_ref, inw2_ref, inb2_ref,
          inw3_ref, inb3_ref, gcomb_ref, ssum_ref, ws_ref, wd_ref,
          fb1_ref, fw2_ref, fb2_ref, fw3_ref, fb3_ref,
          ga_ref, gm_ref, gb1_ref, gw2_ref, gb2_ref, gw3_ref, gb3_ref,
          wih_ref, whh_ref, lb_ref,
          rw1_ref, rb1_ref, rw2_ref, rb2_ref, rw3_ref, rb3_ref,
          out_ref):
    relu = lambda v: jnp.maximum(v, 0.0)

    def mm(a, b):
        return jax.lax.dot_general(a, b, (((1,), (0,)), ((), ())),
                                   preferred_element_type=jnp.float32)

    x = relu(mm(emb_ref[0], w1e_ref[...]) + inb1_ref[...])
    x = relu(mm(x, inw2_ref[...]) + inb2_ref[...])
    x = mm(x, inw3_ref[...]) + inb3_ref[...]

    h = x
    c = c0_ref[0]
    bf = jnp.bfloat16
    for it in range(_ITERS):
        hws = mm(h, ws_ref[...]).astype(bf)
        hwd = mm(h, wd_ref[...]).astype(bf)
        parts = []
        for j in range(_BBLK):
            hcat = jnp.concatenate([hws[j * _NP:(j + 1) * _NP],
                                    hwd[j * _NP:(j + 1) * _NP],
                                    fb1_ref[...]], axis=0)
            parts.append(mm(gcomb_ref[...], hcat))
        a = relu(jnp.concatenate(parts, axis=0))
        a = relu(mm(a, fw2_ref[...]) + fb2_ref[...])
        sa = jnp.concatenate(
            [mm(ssum_ref[...], a[j * _NEP:(j + 1) * _NEP])
             for j in range(_BBLK)], axis=0)
        m = mm(sa, fw3_ref[...]) + fb3_ref[...]

        u = relu(mm(x, ga_ref[...]) + mm(m, gm_ref[...]) + gb1_ref[...])
        u = relu(mm(u, gw2_ref[...]) + gb2_ref[...])
        li = mm(u, gw3_ref[...]) + gb3_ref[...]

        gates = mm(li, wih_ref[...]) + mm(h, whh_ref[...]) + lb_ref[...]
        i_g = gates[:, 0 * _H:1 * _H]
        f_g = gates[:, 1 * _H:2 * _H]
        g_g = gates[:, 2 * _H:3 * _H]
        o_g = gates[:, 3 * _H:4 * _H]
        c = jax.nn.sigmoid(f_g) * c + jax.nn.sigmoid(i_g) * jnp.tanh(g_g)
        h = jax.nn.sigmoid(o_g) * jnp.tanh(c)

        r = relu(mm(h, rw1_ref[...]) + rb1_ref[...])
        r = relu(mm(r, rw2_ref[...]) + rb2_ref[...])
        r = mm(r, rw3_ref[...]) + rb3_ref[...]
        for j in range(_BBLK):
            out_ref[it, j] = r[j * _NP:j * _NP + _N]


def kernel(grids, iters, embed, in_W1, in_b1, in_W2, in_b2, in_W3, in_b3,
           f_W1, f_b1, f_W2, f_b2, f_W3, f_b3,
           g_W1, g_b1, g_W2, g_b2, g_W3, g_b3,
           lstm_Wih, lstm_Whh, lstm_bih, lstm_bhh,
           r_W1, r_b1, r_W2, r_b2, r_W3, r_b3, c0):
    del iters
    f32 = jnp.float32
    row = lambda v: v.reshape(1, -1).astype(f32)

    g4 = grids.astype(jnp.int32).reshape(_B // _BBLK, _BBLK, _N)
    gp = jnp.pad(g4, ((0, 0), (0, 0), (0, _NP - _N)))
    idx = gp.reshape(-1)
    emb = _sc_embed_gather(jnp.pad(embed, ((0, 0), (0, 128 - _E))),
                           idx, idx.shape[0])
    embr = emb.reshape(_B // _BBLK, _M, 128)
    c4 = jnp.pad(c0.reshape(_B, _N, _H), ((0, 0), (0, _NP - _N), (0, 0)))
    c0r = c4.reshape(_B // _BBLK, _M, _H)

    bf16 = jnp.bfloat16
    fb18 = jnp.zeros((8, _H), f32).at[0].set(f_b1).astype(bf16)

    lb = row(lstm_bih + lstm_bhh)
    wih = lstm_Wih.T
    whh = lstm_Whh.T

    args = [embr, c0r, jnp.pad(in_W1, ((0, 128 - _E), (0, 0))),
            row(in_b1), in_W2, row(in_b2), in_W3,
            row(in_b3), jnp.asarray(_GCOMB, bf16), jnp.asarray(_SSUM),
            f_W1[_H:], f_W1[:_H], fb18, f_W2, row(f_b2), f_W3,
            row(f_b3) * _DEG, g_W1[:_H], g_W1[_H:], row(g_b1), g_W2,
            row(g_b2), g_W3, row(g_b3), wih, whh, lb,
            r_W1, row(r_b1), r_W2, row(r_b2), r_W3, row(r_b3)]

    specs = [pl.BlockSpec((1, _M, 128), lambda b: (b, 0, 0)),
             pl.BlockSpec((1, _M, _H), lambda b: (b, 0, 0))]
    specs += [pl.BlockSpec(a.shape, lambda b, n=a.ndim: (0,) * n)
              for a in args[2:]]

    return pl.pallas_call(
        _body,
        grid=(_B // _BBLK,),
        in_specs=specs,
        out_specs=pl.BlockSpec((_ITERS, _BBLK, _N, _OUT),
                               lambda b: (0, b, 0, 0)),
        out_shape=jax.ShapeDtypeStruct((_ITERS, _B, _N, _OUT), f32),
    )(*args)


# speedup vs baseline: 1.5552x; 1.5552x over previous
import itertools

import numpy as np
import jax
import jax.numpy as jnp
from jax.experimental import pallas as pl

_DIM_X, _DIM_Y = 3, 3
_MAX_DIGIT = _DIM_X * _DIM_Y
_N = _MAX_DIGIT * _MAX_DIGIT
_NP = 88
_E = 16
_H = 96
_B = 64
_DEG = 20
_NE = _N * _DEG
_NEP = 1664
_ITERS = 2
_OUT = _MAX_DIGIT
_BBLK = 16
_NCHUNK = _B // _BBLK
_M = _BBLK * _NP


def _edge_table():
    rows = []
    for row in range(_MAX_DIGIT):
        for col in range(_MAX_DIGIT):
            s = {(row, i) for i in range(_MAX_DIGIT)}
            s |= {(i, col) for i in range(_MAX_DIGIT)}
            x_min = row // _DIM_X * _DIM_X
            y_min = col // _DIM_Y * _DIM_Y
            s |= set(itertools.product(range(x_min, x_min + _DIM_X),
                                       range(y_min, y_min + _DIM_Y)))
            s -= {(row, col)}
            rows.append(sorted(r * _MAX_DIGIT + c for r, c in s))
    return np.array(rows, dtype=np.int32)


def _gather_mats():
    edges = _edge_table()
    src = edges.reshape(-1)
    dst = np.repeat(np.arange(_N), _DEG)
    e = np.arange(_NE)
    g = np.zeros((_NEP, 2 * _NP + 8), np.float32)
    g[e, src] = 1.0
    g[e, _NP + dst] = 1.0
    g[e, 2 * _NP] = 1.0
    s = np.zeros((_NP, _NEP), np.float32)
    s[dst, e] = 1.0
    return g, s


_GCOMB, _SSUM = _gather_mats()


def _body(grids_ref, c0_ref, w1e_ref, inw2_ref, inb2_ref,
          inw3_ref, inb3_ref, gcomb_ref, ssum_ref, ws_ref, wd_ref,
          fb1_ref, fw2_ref, fb2_ref, fw3_ref, fb3_ref,
          ga_ref, gm_ref, gb1_ref, gw2_ref, gb2_ref, gw3_ref, gb3_ref,
          wih_ref, whh_ref, lb_ref,
          rw1_ref, rb1_ref, rw2_ref, rb2_ref, rw3_ref, rb3_ref,
          out_ref):
    relu = lambda v: jnp.maximum(v, 0.0)

    def mm(a, b):
        return jax.lax.dot_general(a, b, (((1,), (0,)), ((), ())),
                                   preferred_element_type=jnp.float32)

    g2 = grids_ref[0]
    iot = jax.lax.broadcasted_iota(jnp.int32, (16, _M), 0)
    oh_t = ((iot == g2) | (iot == 10)).astype(jnp.float32)
    x = jax.lax.dot_general(oh_t, w1e_ref[...], (((0,), (0,)), ((), ())),
                            preferred_element_type=jnp.float32)
    x = relu(x)
    x = relu(mm(x, inw2_ref[...]) + inb2_ref[...])
    x = mm(x, inw3_ref[...]) + inb3_ref[...]

    h = x
    c = c0_ref[0]
    bf = jnp.bfloat16
    for it in range(_ITERS):
        hws = mm(h, ws_ref[...]).astype(bf)
        hwd = mm(h, wd_ref[...]).astype(bf)
        parts = []
        for j in range(_BBLK):
            hcat = jnp.concatenate([hws[j * _NP:(j + 1) * _NP],
                                    hwd[j * _NP:(j + 1) * _NP],
                                    fb1_ref[...]], axis=0)
            parts.append(mm(gcomb_ref[...], hcat))
        a = relu(jnp.concatenate(parts, axis=0))
        a = relu(mm(a, fw2_ref[...]) + fb2_ref[...])
        sa = jnp.concatenate(
            [mm(ssum_ref[...], a[j * _NEP:(j + 1) * _NEP])
             for j in range(_BBLK)], axis=0)
        m = mm(sa, fw3_ref[...]) + fb3_ref[...]

        u = relu(mm(x, ga_ref[...]) + mm(m, gm_ref[...]) + gb1_ref[...])
        u = relu(mm(u, gw2_ref[...]) + gb2_ref[...])
        li = mm(u, gw3_ref[...]) + gb3_ref[...]

        gates = mm(li, wih_ref[...]) + mm(h, whh_ref[...]) + lb_ref[...]
        i_g = gates[:, 0 * _H:1 * _H]
        f_g = gates[:, 1 * _H:2 * _H]
        g_g = gates[:, 2 * _H:3 * _H]
        o_g = gates[:, 3 * _H:4 * _H]
        c = jax.nn.sigmoid(f_g) * c + jax.nn.sigmoid(i_g) * jnp.tanh(g_g)
        h = jax.nn.sigmoid(o_g) * jnp.tanh(c)

        r = relu(mm(h, rw1_ref[...]) + rb1_ref[...])
        r = relu(mm(r, rw2_ref[...]) + rb2_ref[...])
        r = mm(r, rw3_ref[...]) + rb3_ref[...]
        for j in range(_BBLK):
            out_ref[it, j] = r[j * _NP:j * _NP + _N]


def kernel(grids, iters, embed, in_W1, in_b1, in_W2, in_b2, in_W3, in_b3,
           f_W1, f_b1, f_W2, f_b2, f_W3, f_b3,
           g_W1, g_b1, g_W2, g_b2, g_W3, g_b3,
           lstm_Wih, lstm_Whh, lstm_bih, lstm_bhh,
           r_W1, r_b1, r_W2, r_b2, r_W3, r_b3, c0):
    del iters
    f32 = jnp.float32
    row = lambda v: v.reshape(1, -1).astype(f32)

    g4 = grids.astype(jnp.int32).reshape(_B // _BBLK, _BBLK, _N)
    gp = jnp.pad(g4, ((0, 0), (0, 0), (0, _NP - _N)), constant_values=15)
    grids3 = gp.reshape(_B // _BBLK, 1, _M)
    c4 = jnp.pad(c0.reshape(_B, _N, _H), ((0, 0), (0, _NP - _N), (0, 0)))
    c0r = c4.reshape(_B // _BBLK, _M, _H)

    bf16 = jnp.bfloat16
    embp = jnp.zeros((16, _E), f32).at[:_MAX_DIGIT + 1].set(embed)
    w1e = (embp @ in_W1).at[10].set(in_b1)
    fb18 = jnp.zeros((8, _H), f32).at[0].set(f_b1).astype(bf16)

    lb = row(lstm_bih + lstm_bhh)
    wih = lstm_Wih.T
    whh = lstm_Whh.T

    args = [grids3, c0r, w1e, in_W2, row(in_b2), in_W3,
            row(in_b3), jnp.asarray(_GCOMB, bf16), jnp.asarray(_SSUM),
            f_W1[_H:], f_W1[:_H], fb18, f_W2, row(f_b2), f_W3,
            row(f_b3) * _DEG, g_W1[:_H], g_W1[_H:], row(g_b1), g_W2,
            row(g_b2), g_W3, row(g_b3), wih, whh, lb,
            r_W1, row(r_b1), r_W2, row(r_b2), r_W3, row(r_b3)]

    specs = [pl.BlockSpec((1, 1, _M), lambda b: (b, 0, 0)),
             pl.BlockSpec((1, _M, _H), lambda b: (b, 0, 0))]
    specs += [pl.BlockSpec(a.shape, lambda b, n=a.ndim: (0,) * n)
              for a in args[2:]]

    return pl.pallas_call(
        _body,
        grid=(_B // _BBLK,),
        in_specs=specs,
        out_specs=pl.BlockSpec((_ITERS, _BBLK, _N, _OUT),
                               lambda b: (0, b, 0, 0)),
        out_shape=jax.ShapeDtypeStruct((_ITERS, _B, _N, _OUT), f32),
    )(*args)
